# baseline (speedup 1.0000x reference)
import jax
import jax.numpy as jnp
from jax import lax
from jax.experimental import pallas as pl
from jax.experimental.pallas import tpu as pltpu

NUM_EMB = 1024
EMB_DIM = 64
COMMIT = 0.25
IMGS = 2
N = 1024 * IMGS
GRID = 16 // IMGS
ROWS = GRID * N


def _vq_body(lat_ref, emb_ref, q_ref, idx_ref, loss_ref, perp_ref,
             sse_mat, iif_sub, m2e_scr, counts_acc, loss_acc):
    step = pl.program_id(0)

    @pl.when(step == 0)
    def _init():
        emb = emb_ref[:]
        m2e_scr[:] = emb * (-2.0)
        sse_col = jnp.sum(emb * emb, axis=1, keepdims=True)
        sse_mat[:] = jnp.broadcast_to(sse_col, (NUM_EMB, N))
        iif_sub[:] = lax.broadcasted_iota(
            jnp.int32, (NUM_EMB, N), 0).astype(jnp.float32)
        counts_acc[:] = jnp.zeros_like(counts_acc)
        loss_acc[0] = 0.0

    latT = jnp.concatenate([lat_ref[i] for i in range(IMGS)], axis=1)
    ssl = latT * latT
    for w in (32, 16, 8, 4, 2, 1):
        ssl = ssl[:w, :] + ssl[w:, :]
    ssl_row = ssl
    mmt = lax.dot_general(m2e_scr[:], latT, (((1,), (0,)), ((), ())),
                          preferred_element_type=jnp.float32)
    d = (sse_mat[:] + ssl_row) + mmt

    md_row = jnp.min(d, axis=0, keepdims=True)
    iif = iif_sub[:]
    idxf = jnp.min(jnp.where(d == md_row, iif, 2.0e9),
                   axis=0, keepdims=True)

    oh = (iif == idxf).astype(jnp.float32)
    qT = lax.dot_general(emb_ref[:], oh, (((0,), (0,)), ((), ())),
                         preferred_element_type=jnp.float32)

    qst = latT + (qT - latT)
    for i in range(IMGS):
        q_ref[i] = qst[:, i * 1024:(i + 1) * 1024]
    idx_ref[0, 0, :] = idxf[0].astype(jnp.int32)

    counts_acc[:] += jnp.sum(oh, axis=1, keepdims=True)
    loss_acc[0] += jnp.sum(md_row)

    @pl.when(step == GRID - 1)
    def _fini():
        loss_ref[0] = COMMIT * loss_acc[0] / float(ROWS * EMB_DIM)
        avg = counts_acc[:] * (1.0 / ROWS)
        perp_ref[0] = jnp.exp(-jnp.sum(avg * jnp.log(avg + 1e-10)))


def kernel(latents_e, embedding_weight):
    B, C, H, W = latents_e.shape
    lat3 = latents_e.reshape(B, C, H * W)

    q3, idx3, loss, perp = pl.pallas_call(
        _vq_body,
        grid=(GRID,),
        in_specs=[
            pl.BlockSpec((IMGS, C, 1024), lambda i: (i, 0, 0)),
            pl.BlockSpec((NUM_EMB, EMB_DIM), lambda i: (0, 0)),
        ],
        out_specs=[
            pl.BlockSpec((IMGS, C, 1024), lambda i: (i, 0, 0)),
            pl.BlockSpec((1, 1, N), lambda i: (i, 0, 0)),
            pl.BlockSpec(memory_space=pltpu.SMEM),
            pl.BlockSpec(memory_space=pltpu.SMEM),
        ],
        out_shape=[
            jax.ShapeDtypeStruct((B, C, 1024), jnp.float32),
            jax.ShapeDtypeStruct((GRID, 1, N), jnp.int32),
            jax.ShapeDtypeStruct((1,), jnp.float32),
            jax.ShapeDtypeStruct((1,), jnp.float32),
        ],
        scratch_shapes=[
            pltpu.VMEM((NUM_EMB, N), jnp.float32),
            pltpu.VMEM((NUM_EMB, N), jnp.float32),
            pltpu.VMEM((NUM_EMB, EMB_DIM), jnp.float32),
            pltpu.VMEM((NUM_EMB, 1), jnp.float32),
            pltpu.SMEM((1,), jnp.float32),
        ],
    )(lat3, embedding_weight)

    return (q3.reshape(B, C, H, W), loss.reshape(()), perp.reshape(()),
            idx3.reshape(B, H * W))


# speedup vs baseline: 1.0070x; 1.0070x over previous
import jax
import jax.numpy as jnp
from jax import lax
from jax.experimental import pallas as pl
from jax.experimental.pallas import tpu as pltpu

NUM_EMB = 1024
EMB_DIM = 64
COMMIT = 0.25
IMGS = 1
N = 1024 * IMGS
GRID = 16 // IMGS
ROWS = GRID * N


def _vq_body(lat_ref, emb_ref, q_ref, idx_ref, loss_ref, perp_ref,
             sse_mat, iif_sub, m2e_scr, counts_acc, loss_acc):
    step = pl.program_id(0)

    @pl.when(step == 0)
    def _init():
        emb = emb_ref[:]
        m2e_scr[:] = emb * (-2.0)
        sse_col = jnp.sum(emb * emb, axis=1, keepdims=True)
        sse_mat[:] = jnp.broadcast_to(sse_col, (NUM_EMB, N))
        iif_sub[:] = lax.broadcasted_iota(
            jnp.int32, (NUM_EMB, N), 0).astype(jnp.float32)
        counts_acc[:] = jnp.zeros_like(counts_acc)
        loss_acc[0] = 0.0

    latT = jnp.concatenate([lat_ref[i] for i in range(IMGS)], axis=1)
    ssl = latT * latT
    for w in (32, 16, 8, 4, 2, 1):
        ssl = ssl[:w, :] + ssl[w:, :]
    ssl_row = ssl
    mmt = lax.dot_general(m2e_scr[:], latT, (((1,), (0,)), ((), ())),
                          preferred_element_type=jnp.float32)
    d = (sse_mat[:] + ssl_row) + mmt

    md_row = jnp.min(d, axis=0, keepdims=True)
    iif = iif_sub[:]
    idxf = jnp.min(jnp.where(d == md_row, iif, 2.0e9),
                   axis=0, keepdims=True)

    oh = (iif == idxf).astype(jnp.float32)
    qT = lax.dot_general(emb_ref[:], oh, (((0,), (0,)), ((), ())),
                         preferred_element_type=jnp.float32)

    qst = latT + (qT - latT)
    for i in range(IMGS):
        q_ref[i] = qst[:, i * 1024:(i + 1) * 1024]
    idx_ref[0, 0, :] = idxf[0].astype(jnp.int32)

    counts_acc[:] += jnp.sum(oh, axis=1, keepdims=True)
    loss_acc[0] += jnp.sum(md_row)

    @pl.when(step == GRID - 1)
    def _fini():
        loss_ref[0] = COMMIT * loss_acc[0] / float(ROWS * EMB_DIM)
        avg = counts_acc[:] * (1.0 / ROWS)
        perp_ref[0] = jnp.exp(-jnp.sum(avg * jnp.log(avg + 1e-10)))


def kernel(latents_e, embedding_weight):
    B, C, H, W = latents_e.shape
    lat3 = latents_e.reshape(B, C, H * W)

    q3, idx3, loss, perp = pl.pallas_call(
        _vq_body,
        grid=(GRID,),
        in_specs=[
            pl.BlockSpec((IMGS, C, 1024), lambda i: (i, 0, 0)),
            pl.BlockSpec((NUM_EMB, EMB_DIM), lambda i: (0, 0)),
        ],
        out_specs=[
            pl.BlockSpec((IMGS, C, 1024), lambda i: (i, 0, 0)),
            pl.BlockSpec((1, 1, N), lambda i: (i, 0, 0)),
            pl.BlockSpec(memory_space=pltpu.SMEM),
            pl.BlockSpec(memory_space=pltpu.SMEM),
        ],
        out_shape=[
            jax.ShapeDtypeStruct((B, C, 1024), jnp.float32),
            jax.ShapeDtypeStruct((GRID, 1, N), jnp.int32),
            jax.ShapeDtypeStruct((1,), jnp.float32),
            jax.ShapeDtypeStruct((1,), jnp.float32),
        ],
        scratch_shapes=[
            pltpu.VMEM((NUM_EMB, N), jnp.float32),
            pltpu.VMEM((NUM_EMB, N), jnp.float32),
            pltpu.VMEM((NUM_EMB, EMB_DIM), jnp.float32),
            pltpu.VMEM((NUM_EMB, 1), jnp.float32),
            pltpu.SMEM((1,), jnp.float32),
        ],
    )(lat3, embedding_weight)

    return (q3.reshape(B, C, H, W), loss.reshape(()), perp.reshape(()),
            idx3.reshape(B, H * W))


# speedup vs baseline: 1.0767x; 1.0693x over previous
import jax
import jax.numpy as jnp
from jax import lax
from jax.experimental import pallas as pl
from jax.experimental.pallas import tpu as pltpu

NUM_EMB = 1024
EMB_DIM = 64
COMMIT = 0.25
IMGS = 1
N = 1024 * IMGS
GRID = 16 // IMGS
ROWS = GRID * N


def _vq_body(lat_ref, emb_ref, q_ref, idx_ref, loss_ref, perp_ref,
             sse_mat, iif_sub, m2e_scr, counts_acc, loss_acc):
    step = pl.program_id(0)

    @pl.when(step == 0)
    def _init():
        emb = emb_ref[:]
        m2e_scr[:] = emb * (-2.0)
        sse_col = jnp.sum(emb * emb, axis=1, keepdims=True)
        sse_mat[:] = jnp.broadcast_to(sse_col, (NUM_EMB, N))
        iif_sub[:] = lax.broadcasted_iota(
            jnp.int32, (NUM_EMB, N), 0).astype(jnp.float32)
        counts_acc[:] = jnp.zeros_like(counts_acc)
        loss_acc[0] = 0.0

    latT = jnp.concatenate([lat_ref[i] for i in range(IMGS)], axis=1)
    ssl = latT * latT
    for w in (32, 16, 8, 4, 2, 1):
        ssl = ssl[:w, :] + ssl[w:, :]
    ssl_row = ssl
    mmt = lax.dot_general(m2e_scr[:], latT, (((1,), (0,)), ((), ())),
                          preferred_element_type=jnp.float32)
    d = (sse_mat[:] + ssl_row) + mmt

    iif = iif_sub[:]
    v, ix = d, iif
    for w in (512, 256, 128, 64, 32, 16, 8, 4, 2, 1):
        lo = v[:w, :] <= v[w:, :]
        v = jnp.where(lo, v[:w, :], v[w:, :])
        ix = jnp.where(lo, ix[:w, :], ix[w:, :])
    md_row, idxf = v, ix

    oh = (iif == idxf).astype(jnp.float32)
    qT = lax.dot_general(emb_ref[:], oh, (((0,), (0,)), ((), ())),
                         preferred_element_type=jnp.float32)

    qst = latT + (qT - latT)
    for i in range(IMGS):
        q_ref[i] = qst[:, i * 1024:(i + 1) * 1024]
    idx_ref[0, 0, :] = idxf[0].astype(jnp.int32)

    counts_acc[:] += jnp.sum(oh, axis=1, keepdims=True)
    loss_acc[0] += jnp.sum(md_row)

    @pl.when(step == GRID - 1)
    def _fini():
        loss_ref[0] = COMMIT * loss_acc[0] / float(ROWS * EMB_DIM)
        avg = counts_acc[:] * (1.0 / ROWS)
        perp_ref[0] = jnp.exp(-jnp.sum(avg * jnp.log(avg + 1e-10)))


def kernel(latents_e, embedding_weight):
    B, C, H, W = latents_e.shape
    lat3 = latents_e.reshape(B, C, H * W)

    q3, idx3, loss, perp = pl.pallas_call(
        _vq_body,
        grid=(GRID,),
        in_specs=[
            pl.BlockSpec((IMGS, C, 1024), lambda i: (i, 0, 0)),
            pl.BlockSpec((NUM_EMB, EMB_DIM), lambda i: (0, 0)),
        ],
        out_specs=[
            pl.BlockSpec((IMGS, C, 1024), lambda i: (i, 0, 0)),
            pl.BlockSpec((1, 1, N), lambda i: (i, 0, 0)),
            pl.BlockSpec(memory_space=pltpu.SMEM),
            pl.BlockSpec(memory_space=pltpu.SMEM),
        ],
        out_shape=[
            jax.ShapeDtypeStruct((B, C, 1024), jnp.float32),
            jax.ShapeDtypeStruct((GRID, 1, N), jnp.int32),
            jax.ShapeDtypeStruct((1,), jnp.float32),
            jax.ShapeDtypeStruct((1,), jnp.float32),
        ],
        scratch_shapes=[
            pltpu.VMEM((NUM_EMB, N), jnp.float32),
            pltpu.VMEM((NUM_EMB, N), jnp.float32),
            pltpu.VMEM((NUM_EMB, EMB_DIM), jnp.float32),
            pltpu.VMEM((NUM_EMB, 1), jnp.float32),
            pltpu.SMEM((1,), jnp.float32),
        ],
    )(lat3, embedding_weight)

    return (q3.reshape(B, C, H, W), loss.reshape(()), perp.reshape(()),
            idx3.reshape(B, H * W))
